# Initial kernel scaffold; baseline (speedup 1.0000x reference)
import jax
import jax.numpy as jnp
from jax.experimental import pallas as pl


def kernel(discrete_data, continuous_data, cat_table, sub_table, ind_table, W1, b1, W2, b2):
    raise NotImplementedError("write your pallas kernel here")


# speedup vs baseline: 2.1205x; 2.1205x over previous
import jax
import jax.numpy as jnp
from jax.experimental import pallas as pl
from jax.experimental.pallas import tpu as pltpu

_DIM = 128
_H = 512


def _body(idx_ref, cat_ref, sub_ref, ind_ref, cont_ref, w1_ref, b1_ref,
          w2_ref, b2_ref, out_ref):
    dn = (((1,), (1,)), ((), ()))
    f32 = jnp.float32
    h = jax.lax.dot_general(cat_ref[0], w1_ref[:, 0:_DIM], dn,
                            preferred_element_type=f32)
    h = h + jax.lax.dot_general(sub_ref[0], w1_ref[:, _DIM:2 * _DIM], dn,
                                preferred_element_type=f32)
    h = h + jax.lax.dot_general(ind_ref[0], w1_ref[:, 2 * _DIM:3 * _DIM], dn,
                                preferred_element_type=f32)
    h = h + jax.lax.dot_general(cont_ref[...], w1_ref[:, 3 * _DIM:3 * _DIM + 3],
                                dn, preferred_element_type=f32)
    h = jnp.maximum(h + b1_ref[...], 0.0)
    out_ref[...] = (jax.lax.dot_general(h, w2_ref[...], dn,
                                        preferred_element_type=f32)
                    + b2_ref[...])


def kernel(discrete_data, continuous_data, cat_table, sub_table, ind_table,
           W1, b1, W2, b2):
    cont = continuous_data.reshape(1, 3)
    b1r = b1.reshape(1, _H)
    b2r = b2.reshape(1, _DIM)
    cat3 = cat_table.reshape(-1, 1, _DIM)
    sub3 = sub_table.reshape(-1, 1, _DIM)
    ind3 = ind_table.reshape(-1, 1, _DIM)
    grid_spec = pltpu.PrefetchScalarGridSpec(
        num_scalar_prefetch=1,
        grid=(1,),
        in_specs=[
            pl.BlockSpec((1, 1, _DIM), lambda i, idx: (idx[0], 0, 0)),
            pl.BlockSpec((1, 1, _DIM), lambda i, idx: (idx[1], 0, 0)),
            pl.BlockSpec((1, 1, _DIM), lambda i, idx: (idx[2], 0, 0)),
            pl.BlockSpec((1, 3), lambda i, idx: (0, 0)),
            pl.BlockSpec((_H, 3 * _DIM + 3), lambda i, idx: (0, 0)),
            pl.BlockSpec((1, _H), lambda i, idx: (0, 0)),
            pl.BlockSpec((_DIM, _H), lambda i, idx: (0, 0)),
            pl.BlockSpec((1, _DIM), lambda i, idx: (0, 0)),
        ],
        out_specs=pl.BlockSpec((1, _DIM), lambda i, idx: (0, 0)),
    )
    out = pl.pallas_call(
        _body,
        grid_spec=grid_spec,
        out_shape=jax.ShapeDtypeStruct((1, _DIM), jnp.float32),
    )(discrete_data, cat3, sub3, ind3, cont, W1, b1r, W2, b2r)
    return out.reshape(_DIM)
